# baseline (speedup 1.0000x reference)
import functools

import jax
import jax.numpy as jnp
from jax import lax
from jax.experimental import pallas as pl
from jax.experimental.pallas import tpu as pltpu
from jax.experimental.pallas import tpu_sc as plsc

N = 10000
E = 320000
D = 128
OUT = 40

NC = 2
NS = 16
NW = NC * NS
CH = 80
NCHUNK = 128
EPW = NCHUNK * CH
E_PAD = NW * EPW
N_PAD = 10240
RPT = N_PAD // NS
DEG_W = 16
NBUF = 4
NRING = 8

_mesh = plsc.VectorSubcoreMesh(core_axis_name="c", subcore_axis_name="s")


@functools.partial(
    pl.kernel,
    out_type=jax.ShapeDtypeStruct((NC, N_PAD, DEG_W), jnp.float32),
    mesh=_mesh,
    scratch_types=[
        pltpu.VMEM((NCHUNK, CH), jnp.int32),
        pltpu.VMEM((CH, DEG_W), jnp.float32),
        pltpu.VMEM_SHARED((N_PAD, DEG_W), jnp.float32),
        pltpu.SemaphoreType.DMA,
    ],
)
def _deg_kernel(dst_hbm, out_hbm, dst_v, ones_v, acc, sem):
    cid = lax.axis_index("c")
    sid = lax.axis_index("s")
    w = cid * NS + sid

    zeros16 = jnp.zeros((16,), jnp.float32)
    ones16 = jnp.ones((16,), jnp.float32)

    def zrow(i, _):
        ones_v[i, :] = zeros16
        return 0

    lax.fori_loop(0, CH, zrow, 0)
    pltpu.sync_copy(dst_hbm.at[w], dst_v)
    for k in range(RPT // CH):
        pltpu.sync_copy(ones_v, acc.at[pl.ds(sid * RPT + k * CH, CH)])

    def fill(i, _):
        ones_v[i, :] = ones16
        return 0

    lax.fori_loop(0, CH, fill, 0)
    plsc.subcore_barrier()

    def body(c, _):
        pltpu.async_copy(ones_v, acc.at[dst_v.at[c]], sem, add=True)
        return 0

    lax.fori_loop(0, NCHUNK, body, 0)

    def drain(c, _):
        pltpu.make_async_copy(ones_v, acc.at[dst_v.at[c]], sem).wait()
        return 0

    lax.fori_loop(0, NCHUNK, drain, 0)
    plsc.subcore_barrier()
    pltpu.sync_copy(acc.at[pl.ds(sid * RPT, RPT)],
                    out_hbm.at[cid, pl.ds(sid * RPT, RPT)])


@functools.partial(
    pl.kernel,
    out_type=jax.ShapeDtypeStruct((NC, N_PAD, D), jnp.float32),
    mesh=_mesh,
    scratch_types=[
        [pltpu.VMEM((CH, D), jnp.float32) for _ in range(NBUF)],
        [pltpu.VMEM((1, CH), jnp.int32) for _ in range(NRING)],
        [pltpu.VMEM((1, CH), jnp.int32) for _ in range(NRING)],
        pltpu.VMEM_SHARED((N_PAD, D), jnp.float32),
        [pltpu.SemaphoreType.DMA for _ in range(NBUF)],
        [pltpu.SemaphoreType.DMA for _ in range(NBUF)],
        [pltpu.SemaphoreType.DMA for _ in range(NRING)],
        [pltpu.SemaphoreType.DMA for _ in range(NRING)],
    ],
)
def _spmm_kernel(ms_hbm, src_hbm, dst_hbm, out_hbm, rbuf, sring, dring, acc,
                 semg, sems, semsr, semid):
    cid = lax.axis_index("c")
    sid = lax.axis_index("s")
    w = cid * NS + sid
    wbase = w * NCHUNK

    zeros16 = jnp.zeros((16,), jnp.float32)

    def zrow(i, _):
        for j in range(D // 16):
            rbuf[0][i, pl.ds(j * 16, 16)] = zeros16
        return 0

    lax.fori_loop(0, CH, zrow, 0)
    for k in range(RPT // CH):
        pltpu.sync_copy(rbuf[0], acc.at[pl.ds(sid * RPT + k * CH, CH)])
    plsc.subcore_barrier()

    def fetch_idx(c, slot):
        pltpu.async_copy(src_hbm.at[pl.ds(wbase + c, 1)], sring[slot],
                         semsr[slot])
        pltpu.async_copy(dst_hbm.at[pl.ds(wbase + c, 1)], dring[slot],
                         semid[slot])

    def wait_src_idx(c, slot):
        pltpu.make_async_copy(src_hbm.at[pl.ds(wbase + c, 1)], sring[slot],
                              semsr[slot]).wait()

    def wait_dst_idx(c, slot):
        pltpu.make_async_copy(dst_hbm.at[pl.ds(wbase + c, 1)], dring[slot],
                              semid[slot]).wait()

    def start_gather(c, slot, buf):
        pltpu.async_copy(ms_hbm.at[sring[slot].at[0]], rbuf[buf], semg[buf])

    def wait_gather(c, slot, buf):
        pltpu.make_async_copy(ms_hbm.at[sring[slot].at[0]], rbuf[buf],
                              semg[buf]).wait()

    def start_scatter(c, slot, buf):
        pltpu.async_copy(rbuf[buf], acc.at[dring[slot].at[0]], sems[buf],
                         add=True)

    def wait_scatter(c, slot, buf):
        pltpu.make_async_copy(rbuf[buf], acc.at[dring[slot].at[0]],
                              sems[buf]).wait()

    for j in range(6):
        fetch_idx(j, j)
    for j in range(2):
        wait_src_idx(j, j)
        start_gather(j, j, j)

    def q_body(q, _):
        c = NRING * q
        for j in range(NRING):
            cj = c + j
            bj = j % NBUF

            @pl.when(cj >= 2)
            def _():
                wait_scatter(cj - 2, (j - 2) % NRING, (j - 2) % NBUF)

            @pl.when(cj + 6 < NCHUNK)
            def _():
                fetch_idx(cj + 6, (j + 6) % NRING)

            wait_gather(cj, j, bj)

            @pl.when(cj + 2 < NCHUNK)
            def _():
                wait_src_idx(cj + 2, (j + 2) % NRING)
                start_gather(cj + 2, (j + 2) % NRING, (j + 2) % NBUF)

            wait_dst_idx(cj, j)
            start_scatter(cj, j, bj)

        return 0

    lax.fori_loop(0, NCHUNK // NRING, q_body, 0)
    wait_scatter(NCHUNK - 2, (NCHUNK - 2) % NRING, (NCHUNK - 2) % NBUF)
    wait_scatter(NCHUNK - 1, (NCHUNK - 1) % NRING, (NCHUNK - 1) % NBUF)
    plsc.subcore_barrier()
    pltpu.sync_copy(acc.at[pl.ds(sid * RPT, RPT)],
                    out_hbm.at[cid, pl.ds(sid * RPT, RPT)])


_RB = 1000


def _dis_from_parts(deg_ref):
    deg = deg_ref[0, :, 0] + deg_ref[1, :, 0] + 1.0
    return lax.rsqrt(deg)


def _tc1_body(x_ref, w1_ref, deg_ref, ms_ref):
    dis = _dis_from_parts(deg_ref)
    h = jnp.dot(x_ref[...], w1_ref[...], preferred_element_type=jnp.float32)
    ms_ref[...] = h * dis[:, None]


def _tc2_body(p_ref, ms1_ref, deg_ref, w2_ref, b1_ref, ms2_ref):
    dis = _dis_from_parts(deg_ref)
    s = p_ref[0] + p_ref[1] + ms1_ref[...]
    h1 = jnp.maximum(s * dis[:, None] + b1_ref[...], 0.0)
    h2 = jnp.dot(h1, w2_ref[...], preferred_element_type=jnp.float32)
    ms2_ref[...] = h2 * dis[:, None]


def _tc3_body(p_ref, ms2_ref, deg_ref, b2_ref, wl_ref, bl_ref, out_ref,
              emb_ref):
    dis = _dis_from_parts(deg_ref)
    s = p_ref[0] + p_ref[1] + ms2_ref[...]
    emb = s * dis[:, None] + b2_ref[...]
    emb_ref[...] = emb
    logits = jnp.dot(emb, wl_ref[...], preferred_element_type=jnp.float32)
    logits = logits + bl_ref[...]
    m = jnp.max(logits, axis=1, keepdims=True)
    z = logits - m
    lse = jnp.log(jnp.sum(jnp.exp(z), axis=1, keepdims=True))
    out_ref[...] = z - lse


def kernel(x, edge_index, W1, b1, W2, b2, Wl, bl):
    npad = E_PAD - E
    srcp = jnp.concatenate(
        [edge_index[0], jnp.zeros((npad,), edge_index.dtype)])
    dstp = jnp.concatenate(
        [edge_index[1], jnp.full((npad,), N_PAD - 1, edge_index.dtype)])
    src = srcp.reshape(NW * NCHUNK, CH)
    dst = dstp.reshape(NW * NCHUNK, CH)
    dst3 = dstp.reshape(NW, NCHUNK, CH)

    deg_parts = _deg_kernel(dst3)

    grid = (N // _RB,)
    full = lambda i: (0, 0)
    rowb = lambda i: (i, 0)
    degb = lambda i: (0, i, 0)

    deg_spec = pl.BlockSpec((NC, _RB, DEG_W), degb)
    part_spec = pl.BlockSpec((NC, _RB, D), degb)
    feat_spec = pl.BlockSpec((_RB, D), rowb)

    ms1 = pl.pallas_call(
        _tc1_body,
        grid=grid,
        in_specs=[feat_spec, pl.BlockSpec((D, D), full), deg_spec],
        out_specs=feat_spec,
        out_shape=jax.ShapeDtypeStruct((N, D), jnp.float32),
    )(x, W1, deg_parts[:, :N, :])

    p1 = _spmm_kernel(ms1, src, dst)

    ms2 = pl.pallas_call(
        _tc2_body,
        grid=grid,
        in_specs=[part_spec, feat_spec, deg_spec,
                  pl.BlockSpec((D, D), full), pl.BlockSpec((1, D), full)],
        out_specs=feat_spec,
        out_shape=jax.ShapeDtypeStruct((N, D), jnp.float32),
    )(p1[:, :N, :], ms1, deg_parts[:, :N, :], W2, b1.reshape(1, D))

    p2 = _spmm_kernel(ms2, src, dst)

    out, emb = pl.pallas_call(
        _tc3_body,
        grid=grid,
        in_specs=[part_spec, feat_spec, deg_spec,
                  pl.BlockSpec((1, D), full), pl.BlockSpec((D, OUT), full),
                  pl.BlockSpec((1, OUT), full)],
        out_specs=[pl.BlockSpec((_RB, OUT), rowb), feat_spec],
        out_shape=[jax.ShapeDtypeStruct((N, OUT), jnp.float32),
                   jax.ShapeDtypeStruct((N, D), jnp.float32)],
    )(p2[:, :N, :], ms2, deg_parts[:, :N, :], b2.reshape(1, D), Wl,
      bl.reshape(1, OUT))

    return (out, emb)
